# baseline (speedup 1.0000x reference)
import functools

import jax
import jax.numpy as jnp
from jax import lax
from jax.experimental import pallas as pl
from jax.experimental.pallas import tpu as pltpu
from jax.experimental.pallas import tpu_sc as plsc


_CH = 200


def _table_body(f_ref, ip_ref, op_ref, l1r_ref, c0_ref, tab_ref, qq_ref):
    pp = jnp.dot(ip_ref[...], l1r_ref[...], preferred_element_type=jnp.float32)
    pad = jnp.zeros((f_ref.shape[0], 32), jnp.float32)
    tab_ref[...] = jnp.concatenate(
        [f_ref[...], pp, pad], axis=1).astype(jnp.bfloat16)
    qq_ref[...] = jnp.dot(op_ref[...], l1r_ref[...],
                          preferred_element_type=jnp.float32) - c0_ref[...]


def _build_table(features, input_pts, output_pts, l1r, c0, NB, C, H1):
    R = 1000
    return pl.pallas_call(
        _table_body,
        grid=(NB // R,),
        in_specs=[
            pl.BlockSpec((R, C), lambda i: (i, 0)),
            pl.BlockSpec((R, 3), lambda i: (i, 0)),
            pl.BlockSpec((R, 3), lambda i: (i, 0)),
            pl.BlockSpec((3, H1), lambda i: (0, 0)),
            pl.BlockSpec((1, H1), lambda i: (0, 0)),
        ],
        out_specs=[
            pl.BlockSpec((R, C + H1 + 32), lambda i: (i, 0)),
            pl.BlockSpec((R, H1), lambda i: (i, 0)),
        ],
        out_shape=[
            jax.ShapeDtypeStruct((NB, C + H1 + 32), jnp.bfloat16),
            jax.ShapeDtypeStruct((NB, H1), jnp.float32),
        ],
    )(features, input_pts, output_pts, l1r, c0)


def _sc_gather_body(table_hbm, idx_hbm, out_hbm, idx_v, buf0, buf1,
                    sem0, sem1, *, per_w, nc):
    wid = lax.axis_index("s") * nc + lax.axis_index("c")
    base = wid * per_w
    pltpu.sync_copy(idx_hbm.at[pl.ds(base, per_w)], idx_v)

    nch = per_w // _CH

    def start(k, buf, sem):
        pltpu.async_copy(
            table_hbm.at[idx_v.at[pl.ds(k * _CH, _CH)]], buf, sem)

    def drain(k, buf, sem):
        pltpu.make_async_copy(
            table_hbm.at[idx_v.at[pl.ds(k * _CH, _CH)]], buf, sem).wait()
        pltpu.sync_copy(buf, out_hbm.at[pl.ds(base + k * _CH, _CH)])

    start(0, buf0, sem0)

    @pl.loop(1, nch - 1, step=2)
    def _pair(k):
        start(k, buf1, sem1)
        drain(k - 1, buf0, sem0)
        start(k + 1, buf0, sem0)
        drain(k, buf1, sem1)

    drain(nch - 1, buf0, sem0)


def _sc_gather(table, idx, n_edges, width):
    info = plsc.get_sparse_core_info()
    nc, ns = info.num_cores, info.num_subcores
    nw = nc * ns
    per_w = n_edges // nw
    assert n_edges % nw == 0 and per_w % _CH == 0 and (per_w // _CH) % 2 == 1
    mesh = plsc.VectorSubcoreMesh(core_axis_name="c", subcore_axis_name="s")
    body = functools.partial(_sc_gather_body, per_w=per_w, nc=nc)
    f = pl.kernel(
        body,
        out_type=jax.ShapeDtypeStruct((n_edges, width), table.dtype),
        mesh=mesh,
        scratch_types=[
            pltpu.VMEM((per_w,), jnp.int32),
            pltpu.VMEM((_CH, width), table.dtype),
            pltpu.VMEM((_CH, width), table.dtype),
            pltpu.SemaphoreType.DMA,
            pltpu.SemaphoreType.DMA,
        ],
        compiler_params=pltpu.CompilerParams(use_tc_tiling_on_sc=False),
    )
    return f(table, idx)


def _block_kernel(g_ref, qq_ref, l2w_ref, l2b_ref, l3w_ref, l3b_ref,
                  wp_ref, bias_ref, out_ref, *, P, K, C, KSZ, OUT, H1):
    PK = P * K
    pp = g_ref[:, C:C + H1].astype(jnp.float32)
    qq = qq_ref[...]
    qq_rep = jnp.broadcast_to(qq[:, None, :], (P, K, H1)).reshape(PK, H1)
    h = jnp.maximum(pp - qq_rep, 0.0)
    h = jnp.maximum(jnp.dot(h, l2w_ref[...],
                            preferred_element_type=jnp.float32)
                    + l2b_ref[...], 0.0)
    d = jnp.maximum(jnp.dot(h, l3w_ref[...],
                            preferred_element_type=jnp.float32)
                    + l3b_ref[...], 0.0)

    feats2 = jnp.tile(g_ref[:, :C].astype(jnp.float32), (1, 2))
    row16 = lax.broadcasted_iota(jnp.int32, (KSZ, 2 * C), 0)
    half = lax.broadcasted_iota(jnp.int32, (KSZ, 2 * C), 1) // C
    acc8 = jnp.zeros((P * 8, OUT), dtype=jnp.float32)
    for i in range(KSZ // 2):
        sel = (row16 == 2 * i + half).astype(jnp.float32)
        dexp = jnp.dot(d, sel, preferred_element_type=jnp.float32)
        prod = feats2 * dexp
        s8 = prod.reshape(P, 2, 8, 2 * C).sum(axis=1).reshape(P * 8, 2 * C)
        acc8 = acc8 + jnp.dot(s8, wp_ref[i],
                              preferred_element_type=jnp.float32)
    out = acc8.reshape(P, 8, OUT).sum(axis=1) * (1.0 / K) + bias_ref[...]
    out_ref[...] = jnp.maximum(out, 0.0)


def kernel(features, input_pts, output_pts, W, bias, centers,
           l1_w, l1_b, l2_w, l2_b, l3_w, l3_b, indices_, neighbor_num):
    B, N, C = features.shape
    K = indices_.shape[2]
    KSZ = centers.shape[1]
    OUT = W.shape[2]
    H1 = l1_w.shape[0]
    NB = B * N
    WIDTH = C + H1 + 32

    l1w3 = l1_w.reshape(H1, 3, KSZ)
    l1r = jnp.sum(l1w3, axis=2).T
    c0 = (l1_b - jnp.einsum('mij,ij->m', l1w3, centers)).reshape(1, H1)

    offs = (jnp.arange(B, dtype=indices_.dtype) * N)[:, None, None]
    idx = (indices_ + offs).reshape(NB * K).astype(jnp.int32)

    table, qq = _build_table(features.reshape(NB, C), input_pts.reshape(NB, 3),
                             output_pts.reshape(NB, 3), l1r, c0, NB, C, H1)
    g = _sc_gather(table, idx, NB * K, WIDTH)

    P = 200
    assert NB % P == 0
    grid = NB // P

    wp = jnp.transpose(W, (1, 0, 2)).reshape(KSZ // 2, 2 * C, OUT)
    body = functools.partial(_block_kernel, P=P, K=K, C=C, KSZ=KSZ,
                             OUT=OUT, H1=H1)
    out = pl.pallas_call(
        body,
        grid=(grid,),
        in_specs=[
            pl.BlockSpec((P * K, WIDTH), lambda i: (i, 0)),
            pl.BlockSpec((P, H1), lambda i: (i, 0)),
            pl.BlockSpec((2 * KSZ, KSZ), lambda i: (0, 0)),
            pl.BlockSpec((1, KSZ), lambda i: (0, 0)),
            pl.BlockSpec((KSZ, KSZ), lambda i: (0, 0)),
            pl.BlockSpec((1, KSZ), lambda i: (0, 0)),
            pl.BlockSpec((KSZ // 2, 2 * C, OUT), lambda i: (0, 0, 0)),
            pl.BlockSpec((1, OUT), lambda i: (0, 0)),
        ],
        out_specs=pl.BlockSpec((P, OUT), lambda i: (i, 0)),
        out_shape=jax.ShapeDtypeStruct((NB, OUT), jnp.float32),
    )(g, qq, l2_w.T, l2_b.reshape(1, -1), l3_w.T, l3_b.reshape(1, -1),
      wp, bias.reshape(1, -1))
    return out.reshape(B, N, OUT)


# speedup vs baseline: 1.0844x; 1.0844x over previous
import functools

import jax
import jax.numpy as jnp
from jax import lax
from jax.experimental import pallas as pl
from jax.experimental.pallas import tpu as pltpu
from jax.experimental.pallas import tpu_sc as plsc


_CH = 200


def _table_body(f_ref, ip_ref, op_ref, l1r_ref, c0_ref, tab_ref, qq_ref):
    pp = jnp.dot(ip_ref[...], l1r_ref[...], preferred_element_type=jnp.float32)
    tab_ref[...] = jnp.concatenate(
        [f_ref[...], pp], axis=1).astype(jnp.bfloat16)
    qq_ref[...] = jnp.dot(op_ref[...], l1r_ref[...],
                          preferred_element_type=jnp.float32) - c0_ref[...]


def _build_table(features, input_pts, output_pts, l1r, c0, NB, C, H1):
    R = 1000
    return pl.pallas_call(
        _table_body,
        grid=(NB // R,),
        in_specs=[
            pl.BlockSpec((R, C), lambda i: (i, 0)),
            pl.BlockSpec((R, 3), lambda i: (i, 0)),
            pl.BlockSpec((R, 3), lambda i: (i, 0)),
            pl.BlockSpec((3, H1), lambda i: (0, 0)),
            pl.BlockSpec((1, H1), lambda i: (0, 0)),
        ],
        out_specs=[
            pl.BlockSpec((R, C + H1), lambda i: (i, 0)),
            pl.BlockSpec((R, H1), lambda i: (i, 0)),
        ],
        out_shape=[
            jax.ShapeDtypeStruct((NB, C + H1), jnp.bfloat16),
            jax.ShapeDtypeStruct((NB, H1), jnp.float32),
        ],
    )(features, input_pts, output_pts, l1r, c0)


def _sc_gather_body(table_hbm, idx_hbm, out_hbm, idx_v, buf0, buf1,
                    sem0, sem1, *, per_w, nc):
    wid = lax.axis_index("s") * nc + lax.axis_index("c")
    base = wid * per_w
    pltpu.sync_copy(idx_hbm.at[pl.ds(base, per_w)], idx_v)

    nch = per_w // _CH

    def start(k, buf, sem):
        pltpu.async_copy(
            table_hbm.at[idx_v.at[pl.ds(k * _CH, _CH)]], buf, sem)

    def drain(k, buf, sem):
        pltpu.make_async_copy(
            table_hbm.at[idx_v.at[pl.ds(k * _CH, _CH)]], buf, sem).wait()
        pltpu.sync_copy(buf, out_hbm.at[pl.ds(base + k * _CH, _CH)])

    start(0, buf0, sem0)

    @pl.loop(1, nch - 1, step=2)
    def _pair(k):
        start(k, buf1, sem1)
        drain(k - 1, buf0, sem0)
        start(k + 1, buf0, sem0)
        drain(k, buf1, sem1)

    drain(nch - 1, buf0, sem0)


def _sc_gather(table, idx, n_edges, width):
    info = plsc.get_sparse_core_info()
    nc, ns = info.num_cores, info.num_subcores
    nw = nc * ns
    per_w = n_edges // nw
    assert n_edges % nw == 0 and per_w % _CH == 0 and (per_w // _CH) % 2 == 1
    mesh = plsc.VectorSubcoreMesh(core_axis_name="c", subcore_axis_name="s")
    body = functools.partial(_sc_gather_body, per_w=per_w, nc=nc)
    f = pl.kernel(
        body,
        out_type=jax.ShapeDtypeStruct((n_edges, width), table.dtype),
        mesh=mesh,
        scratch_types=[
            pltpu.VMEM((per_w,), jnp.int32),
            pltpu.VMEM((_CH, width), table.dtype),
            pltpu.VMEM((_CH, width), table.dtype),
            pltpu.SemaphoreType.DMA,
            pltpu.SemaphoreType.DMA,
        ],
        compiler_params=pltpu.CompilerParams(use_tc_tiling_on_sc=False),
    )
    return f(table, idx)


def _block_kernel(g_ref, qq_ref, l2w_ref, l2b_ref, l3w_ref, l3b_ref,
                  wp_ref, bias_ref, out_ref, *, P, K, C, KSZ, OUT, H1):
    PK = P * K
    pp = g_ref[:, C:C + H1].astype(jnp.float32)
    qq = qq_ref[...]
    qq_rep = jnp.broadcast_to(qq[:, None, :], (P, K, H1)).reshape(PK, H1)
    h = jnp.maximum(pp - qq_rep, 0.0)
    h = jnp.maximum(jnp.dot(h, l2w_ref[...],
                            preferred_element_type=jnp.float32)
                    + l2b_ref[...], 0.0)
    d = jnp.maximum(jnp.dot(h, l3w_ref[...],
                            preferred_element_type=jnp.float32)
                    + l3b_ref[...], 0.0)

    feats2 = jnp.tile(g_ref[:, :C].astype(jnp.float32), (1, 2))
    row16 = lax.broadcasted_iota(jnp.int32, (KSZ, 2 * C), 0)
    half = lax.broadcasted_iota(jnp.int32, (KSZ, 2 * C), 1) // C
    acc8 = jnp.zeros((P * 8, OUT), dtype=jnp.float32)
    for i in range(KSZ // 2):
        sel = (row16 == 2 * i + half).astype(jnp.float32)
        dexp = jnp.dot(d, sel, preferred_element_type=jnp.float32)
        prod = feats2 * dexp
        s8 = prod.reshape(P, 2, 8, 2 * C).sum(axis=1).reshape(P * 8, 2 * C)
        acc8 = acc8 + jnp.dot(s8, wp_ref[i],
                              preferred_element_type=jnp.float32)
    out = acc8.reshape(P, 8, OUT).sum(axis=1) * (1.0 / K) + bias_ref[...]
    out_ref[...] = jnp.maximum(out, 0.0)


def kernel(features, input_pts, output_pts, W, bias, centers,
           l1_w, l1_b, l2_w, l2_b, l3_w, l3_b, indices_, neighbor_num):
    B, N, C = features.shape
    K = indices_.shape[2]
    KSZ = centers.shape[1]
    OUT = W.shape[2]
    H1 = l1_w.shape[0]
    NB = B * N
    WIDTH = C + H1

    l1w3 = l1_w.reshape(H1, 3, KSZ)
    l1r = jnp.sum(l1w3, axis=2).T
    c0 = (l1_b - jnp.einsum('mij,ij->m', l1w3, centers)).reshape(1, H1)

    offs = (jnp.arange(B, dtype=indices_.dtype) * N)[:, None, None]
    idx = (indices_ + offs).reshape(NB * K).astype(jnp.int32)

    table, qq = _build_table(features.reshape(NB, C), input_pts.reshape(NB, 3),
                             output_pts.reshape(NB, 3), l1r, c0, NB, C, H1)
    g = _sc_gather(table, idx, NB * K, WIDTH)

    P = 400
    assert NB % P == 0
    grid = NB // P

    wp = jnp.transpose(W, (1, 0, 2)).reshape(KSZ // 2, 2 * C, OUT)
    body = functools.partial(_block_kernel, P=P, K=K, C=C, KSZ=KSZ,
                             OUT=OUT, H1=H1)
    out = pl.pallas_call(
        body,
        grid=(grid,),
        in_specs=[
            pl.BlockSpec((P * K, WIDTH), lambda i: (i, 0)),
            pl.BlockSpec((P, H1), lambda i: (i, 0)),
            pl.BlockSpec((2 * KSZ, KSZ), lambda i: (0, 0)),
            pl.BlockSpec((1, KSZ), lambda i: (0, 0)),
            pl.BlockSpec((KSZ, KSZ), lambda i: (0, 0)),
            pl.BlockSpec((1, KSZ), lambda i: (0, 0)),
            pl.BlockSpec((KSZ // 2, 2 * C, OUT), lambda i: (0, 0, 0)),
            pl.BlockSpec((1, OUT), lambda i: (0, 0)),
        ],
        out_specs=pl.BlockSpec((P, OUT), lambda i: (i, 0)),
        out_shape=jax.ShapeDtypeStruct((NB, OUT), jnp.float32),
    )(g, qq, l2_w.T, l2_b.reshape(1, -1), l3_w.T, l3_b.reshape(1, -1),
      wp, bias.reshape(1, -1))
    return out.reshape(B, N, OUT)


# speedup vs baseline: 1.0867x; 1.0021x over previous
import functools

import jax
import jax.numpy as jnp
from jax import lax
from jax.experimental import pallas as pl
from jax.experimental.pallas import tpu as pltpu
from jax.experimental.pallas import tpu_sc as plsc


_CH = 200


def _table_body(f_ref, ip_ref, op_ref, l1r_ref, c0_ref, tab_ref, qq_ref):
    pp = jnp.dot(ip_ref[...], l1r_ref[...], preferred_element_type=jnp.float32)
    tab_ref[...] = jnp.concatenate(
        [f_ref[...], pp], axis=1).astype(jnp.bfloat16)
    qq_ref[...] = jnp.dot(op_ref[...], l1r_ref[...],
                          preferred_element_type=jnp.float32) - c0_ref[...]


def _build_table(features, input_pts, output_pts, l1r, c0, NB, C, H1):
    R = 1000
    return pl.pallas_call(
        _table_body,
        grid=(NB // R,),
        in_specs=[
            pl.BlockSpec((R, C), lambda i: (i, 0)),
            pl.BlockSpec((R, 3), lambda i: (i, 0)),
            pl.BlockSpec((R, 3), lambda i: (i, 0)),
            pl.BlockSpec((3, H1), lambda i: (0, 0)),
            pl.BlockSpec((1, H1), lambda i: (0, 0)),
        ],
        out_specs=[
            pl.BlockSpec((R, C + H1), lambda i: (i, 0)),
            pl.BlockSpec((R, H1), lambda i: (i, 0)),
        ],
        out_shape=[
            jax.ShapeDtypeStruct((NB, C + H1), jnp.bfloat16),
            jax.ShapeDtypeStruct((NB, H1), jnp.float32),
        ],
    )(features, input_pts, output_pts, l1r, c0)


def _sc_gather_body(table_hbm, idx_hbm, out_hbm, idx_v, buf0, buf1,
                    sem0, sem1, *, per_w, nc):
    wid = lax.axis_index("s") * nc + lax.axis_index("c")
    base = wid * per_w
    pltpu.sync_copy(idx_hbm.at[pl.ds(base, per_w)], idx_v)

    nch = per_w // _CH

    def start(k, buf, sem):
        pltpu.async_copy(
            table_hbm.at[idx_v.at[pl.ds(k * _CH, _CH)]], buf, sem)

    def drain(k, buf, sem):
        pltpu.make_async_copy(
            table_hbm.at[idx_v.at[pl.ds(k * _CH, _CH)]], buf, sem).wait()
        pltpu.sync_copy(buf, out_hbm.at[pl.ds(base + k * _CH, _CH)])

    start(0, buf0, sem0)

    @pl.loop(1, nch - 1, step=2)
    def _pair(k):
        start(k, buf1, sem1)
        drain(k - 1, buf0, sem0)
        start(k + 1, buf0, sem0)
        drain(k, buf1, sem1)

    drain(nch - 1, buf0, sem0)


def _sc_gather(table, idx, n_edges, width):
    info = plsc.get_sparse_core_info()
    nc, ns = info.num_cores, info.num_subcores
    nw = nc * ns
    per_w = n_edges // nw
    assert n_edges % nw == 0 and per_w % _CH == 0 and (per_w // _CH) % 2 == 1
    mesh = plsc.VectorSubcoreMesh(core_axis_name="c", subcore_axis_name="s")
    body = functools.partial(_sc_gather_body, per_w=per_w, nc=nc)
    f = pl.kernel(
        body,
        out_type=jax.ShapeDtypeStruct((n_edges, width), table.dtype),
        mesh=mesh,
        scratch_types=[
            pltpu.VMEM((per_w,), jnp.int32),
            pltpu.VMEM((_CH, width), table.dtype),
            pltpu.VMEM((_CH, width), table.dtype),
            pltpu.SemaphoreType.DMA,
            pltpu.SemaphoreType.DMA,
        ],
        compiler_params=pltpu.CompilerParams(use_tc_tiling_on_sc=False),
    )
    return f(table, idx)


def _block_kernel(g_ref, qq_ref, l2w_ref, l2b_ref, l3w_ref, l3b_ref,
                  wp_ref, bias_ref, out_ref, *, P, K, C, KSZ, OUT, H1):
    PK = P * K
    pp = g_ref[:, C:C + H1].astype(jnp.float32)
    qq = qq_ref[...]
    qq_rep = jnp.broadcast_to(qq[:, None, :], (P, K, H1)).reshape(PK, H1)
    h = jnp.maximum(pp - qq_rep, 0.0)
    h = jnp.maximum(jnp.dot(h, l2w_ref[...],
                            preferred_element_type=jnp.float32)
                    + l2b_ref[...], 0.0)
    d = jnp.maximum(jnp.dot(h, l3w_ref[...],
                            preferred_element_type=jnp.float32)
                    + l3b_ref[...], 0.0)

    feats2 = jnp.tile(g_ref[:, :C], (1, 2))
    row16 = lax.broadcasted_iota(jnp.int32, (KSZ, 2 * C), 0)
    half = lax.broadcasted_iota(jnp.int32, (KSZ, 2 * C), 1) // C
    db = d.astype(jnp.bfloat16)
    acc8 = jnp.zeros((P * 8, OUT), dtype=jnp.float32)
    for i in range(KSZ // 2):
        sel = (row16 == 2 * i + half).astype(jnp.bfloat16)
        dexp = jnp.dot(db, sel,
                       preferred_element_type=jnp.float32).astype(jnp.bfloat16)
        prod = feats2 * dexp
        s8 = prod.reshape(P, 2, 8, 2 * C).sum(axis=1).reshape(P * 8, 2 * C)
        acc8 = acc8 + jnp.dot(s8, wp_ref[i],
                              preferred_element_type=jnp.float32)
    out = acc8.reshape(P, 8, OUT).sum(axis=1) * (1.0 / K) + bias_ref[...]
    out_ref[...] = jnp.maximum(out, 0.0)


def kernel(features, input_pts, output_pts, W, bias, centers,
           l1_w, l1_b, l2_w, l2_b, l3_w, l3_b, indices_, neighbor_num):
    B, N, C = features.shape
    K = indices_.shape[2]
    KSZ = centers.shape[1]
    OUT = W.shape[2]
    H1 = l1_w.shape[0]
    NB = B * N
    WIDTH = C + H1

    l1w3 = l1_w.reshape(H1, 3, KSZ)
    l1r = jnp.sum(l1w3, axis=2).T
    c0 = (l1_b - jnp.einsum('mij,ij->m', l1w3, centers)).reshape(1, H1)

    offs = (jnp.arange(B, dtype=indices_.dtype) * N)[:, None, None]
    idx = (indices_ + offs).reshape(NB * K).astype(jnp.int32)

    table, qq = _build_table(features.reshape(NB, C), input_pts.reshape(NB, 3),
                             output_pts.reshape(NB, 3), l1r, c0, NB, C, H1)
    g = _sc_gather(table, idx, NB * K, WIDTH)

    P = 400
    assert NB % P == 0
    grid = NB // P

    wp = jnp.transpose(W, (1, 0, 2)).reshape(
        KSZ // 2, 2 * C, OUT).astype(jnp.bfloat16)
    body = functools.partial(_block_kernel, P=P, K=K, C=C, KSZ=KSZ,
                             OUT=OUT, H1=H1)
    out = pl.pallas_call(
        body,
        grid=(grid,),
        in_specs=[
            pl.BlockSpec((P * K, WIDTH), lambda i: (i, 0)),
            pl.BlockSpec((P, H1), lambda i: (i, 0)),
            pl.BlockSpec((2 * KSZ, KSZ), lambda i: (0, 0)),
            pl.BlockSpec((1, KSZ), lambda i: (0, 0)),
            pl.BlockSpec((KSZ, KSZ), lambda i: (0, 0)),
            pl.BlockSpec((1, KSZ), lambda i: (0, 0)),
            pl.BlockSpec((KSZ // 2, 2 * C, OUT), lambda i: (0, 0, 0)),
            pl.BlockSpec((1, OUT), lambda i: (0, 0)),
        ],
        out_specs=pl.BlockSpec((P, OUT), lambda i: (i, 0)),
        out_shape=jax.ShapeDtypeStruct((NB, OUT), jnp.float32),
    )(g, qq, l2_w.T, l2_b.reshape(1, -1), l3_w.T, l3_b.reshape(1, -1),
      wp, bias.reshape(1, -1))
    return out.reshape(B, N, OUT)


# speedup vs baseline: 1.5660x; 1.4410x over previous
import functools

import jax
import jax.numpy as jnp
from jax import lax
from jax.experimental import pallas as pl
from jax.experimental.pallas import tpu as pltpu
from jax.experimental.pallas import tpu_sc as plsc


_CH = 200


def _table_body(f_ref, ip_ref, op_ref, l1r_ref, c0_ref, tab_ref, qq_ref):
    pp = jnp.dot(ip_ref[...], l1r_ref[...], preferred_element_type=jnp.float32)
    pad = jnp.zeros((f_ref.shape[0], 32), jnp.float32)
    tab_ref[...] = jnp.concatenate([f_ref[...], pp, pad], axis=1)
    qq_ref[...] = jnp.dot(op_ref[...], l1r_ref[...],
                          preferred_element_type=jnp.float32) - c0_ref[...]


def _build_table(features, input_pts, output_pts, l1r, c0, NB, C, H1):
    R = 1000
    return pl.pallas_call(
        _table_body,
        grid=(NB // R,),
        in_specs=[
            pl.BlockSpec((R, C), lambda i: (i, 0)),
            pl.BlockSpec((R, 3), lambda i: (i, 0)),
            pl.BlockSpec((R, 3), lambda i: (i, 0)),
            pl.BlockSpec((3, H1), lambda i: (0, 0)),
            pl.BlockSpec((1, H1), lambda i: (0, 0)),
        ],
        out_specs=[
            pl.BlockSpec((R, C + H1 + 32), lambda i: (i, 0)),
            pl.BlockSpec((R, H1), lambda i: (i, 0)),
        ],
        out_shape=[
            jax.ShapeDtypeStruct((NB, C + H1 + 32), jnp.float32),
            jax.ShapeDtypeStruct((NB, H1), jnp.float32),
        ],
    )(features, input_pts, output_pts, l1r, c0)


def _sc_gather_body(table_hbm, idx_hbm, out_hbm, idx_v, buf0, buf1,
                    sem0, sem1, *, per_w, nc):
    wid = lax.axis_index("s") * nc + lax.axis_index("c")
    base = wid * per_w
    pltpu.sync_copy(idx_hbm.at[pl.ds(base, per_w)], idx_v)

    nch = per_w // _CH

    def start(k, buf, sem):
        pltpu.async_copy(
            table_hbm.at[idx_v.at[pl.ds(k * _CH, _CH)]], buf, sem)

    def drain(k, buf, sem):
        pltpu.make_async_copy(
            table_hbm.at[idx_v.at[pl.ds(k * _CH, _CH)]], buf, sem).wait()
        pltpu.sync_copy(buf, out_hbm.at[pl.ds(base + k * _CH, _CH)])

    start(0, buf0, sem0)

    @pl.loop(1, nch - 1, step=2)
    def _pair(k):
        start(k, buf1, sem1)
        drain(k - 1, buf0, sem0)
        start(k + 1, buf0, sem0)
        drain(k, buf1, sem1)

    drain(nch - 1, buf0, sem0)


def _sc_gather(table, idx, n_edges, width):
    info = plsc.get_sparse_core_info()
    nc, ns = info.num_cores, info.num_subcores
    nw = nc * ns
    per_w = n_edges // nw
    assert n_edges % nw == 0 and per_w % _CH == 0 and (per_w // _CH) % 2 == 1
    mesh = plsc.VectorSubcoreMesh(core_axis_name="c", subcore_axis_name="s")
    body = functools.partial(_sc_gather_body, per_w=per_w, nc=nc)
    f = pl.kernel(
        body,
        out_type=jax.ShapeDtypeStruct((n_edges, width), table.dtype),
        mesh=mesh,
        scratch_types=[
            pltpu.VMEM((per_w,), jnp.int32),
            pltpu.VMEM((_CH, width), table.dtype),
            pltpu.VMEM((_CH, width), table.dtype),
            pltpu.SemaphoreType.DMA,
            pltpu.SemaphoreType.DMA,
        ],
    )
    return f(table, idx)


def _block_kernel(g_ref, qq_ref, l2w_ref, l2b_ref, l3w_ref, l3b_ref,
                  wp_ref, bias_ref, out_ref, *, P, K, C, KSZ, OUT, H1):
    PK = P * K
    pp = g_ref[:, C:C + H1]
    qq = qq_ref[...]
    qq_rep = jnp.broadcast_to(qq[:, None, :], (P, K, H1)).reshape(PK, H1)
    h = jnp.maximum(pp - qq_rep, 0.0)
    h = jnp.maximum(jnp.dot(h, l2w_ref[...],
                            preferred_element_type=jnp.float32)
                    + l2b_ref[...], 0.0)
    d = jnp.maximum(jnp.dot(h, l3w_ref[...],
                            preferred_element_type=jnp.float32)
                    + l3b_ref[...], 0.0)

    feats2 = jnp.tile(g_ref[:, :C], (1, 2))
    row16 = lax.broadcasted_iota(jnp.int32, (KSZ, 2 * C), 0)
    half = lax.broadcasted_iota(jnp.int32, (KSZ, 2 * C), 1) // C
    acc8 = jnp.zeros((P * 8, OUT), dtype=jnp.float32)
    for i in range(KSZ // 2):
        sel = (row16 == 2 * i + half).astype(jnp.float32)
        dexp = jnp.dot(d, sel, preferred_element_type=jnp.float32)
        prod = feats2 * dexp
        s8 = prod.reshape(P, 2, 8, 2 * C).sum(axis=1).reshape(P * 8, 2 * C)
        acc8 = acc8 + jnp.dot(s8, wp_ref[i],
                              preferred_element_type=jnp.float32)
    out = acc8.reshape(P, 8, OUT).sum(axis=1) * (1.0 / K) + bias_ref[...]
    out_ref[...] = jnp.maximum(out, 0.0)


def kernel(features, input_pts, output_pts, W, bias, centers,
           l1_w, l1_b, l2_w, l2_b, l3_w, l3_b, indices_, neighbor_num):
    B, N, C = features.shape
    K = indices_.shape[2]
    KSZ = centers.shape[1]
    OUT = W.shape[2]
    H1 = l1_w.shape[0]
    NB = B * N
    WIDTH = C + H1 + 32

    l1w3 = l1_w.reshape(H1, 3, KSZ)
    l1r = jnp.sum(l1w3, axis=2).T
    c0 = (l1_b - jnp.einsum('mij,ij->m', l1w3, centers)).reshape(1, H1)

    offs = (jnp.arange(B, dtype=indices_.dtype) * N)[:, None, None]
    idx = (indices_ + offs).reshape(NB * K).astype(jnp.int32)

    table, qq = _build_table(features.reshape(NB, C), input_pts.reshape(NB, 3),
                             output_pts.reshape(NB, 3), l1r, c0, NB, C, H1)
    g = _sc_gather(table, idx, NB * K, WIDTH)

    P = 400
    assert NB % P == 0
    grid = NB // P

    wp = jnp.transpose(W, (1, 0, 2)).reshape(KSZ // 2, 2 * C, OUT)
    body = functools.partial(_block_kernel, P=P, K=K, C=C, KSZ=KSZ,
                             OUT=OUT, H1=H1)
    out = pl.pallas_call(
        body,
        grid=(grid,),
        in_specs=[
            pl.BlockSpec((P * K, WIDTH), lambda i: (i, 0)),
            pl.BlockSpec((P, H1), lambda i: (i, 0)),
            pl.BlockSpec((2 * KSZ, KSZ), lambda i: (0, 0)),
            pl.BlockSpec((1, KSZ), lambda i: (0, 0)),
            pl.BlockSpec((KSZ, KSZ), lambda i: (0, 0)),
            pl.BlockSpec((1, KSZ), lambda i: (0, 0)),
            pl.BlockSpec((KSZ // 2, 2 * C, OUT), lambda i: (0, 0, 0)),
            pl.BlockSpec((1, OUT), lambda i: (0, 0)),
        ],
        out_specs=pl.BlockSpec((P, OUT), lambda i: (i, 0)),
        out_shape=jax.ShapeDtypeStruct((NB, OUT), jnp.float32),
    )(g, qq, l2_w.T, l2_b.reshape(1, -1), l3_w.T, l3_b.reshape(1, -1),
      wp, bias.reshape(1, -1))
    return out.reshape(B, N, OUT)


# speedup vs baseline: 1.6827x; 1.0745x over previous
import functools

import jax
import jax.numpy as jnp
from jax import lax
from jax.experimental import pallas as pl
from jax.experimental.pallas import tpu as pltpu
from jax.experimental.pallas import tpu_sc as plsc


_CH = 200


def _table_body(f_ref, ip_ref, op_ref, l1r_ref, c0_ref, tab_ref, qq_ref):
    pp = jnp.dot(ip_ref[...], l1r_ref[...], preferred_element_type=jnp.float32)
    pad = jnp.zeros((f_ref.shape[0], 32), jnp.float32)
    tab_ref[...] = jnp.concatenate([f_ref[...], pp, pad], axis=1)
    qq_ref[...] = jnp.dot(op_ref[...], l1r_ref[...],
                          preferred_element_type=jnp.float32) - c0_ref[...]


def _build_table(features, input_pts, output_pts, l1r, c0, NB, C, H1):
    R = 1000
    return pl.pallas_call(
        _table_body,
        grid=(NB // R,),
        in_specs=[
            pl.BlockSpec((R, C), lambda i: (i, 0)),
            pl.BlockSpec((R, 3), lambda i: (i, 0)),
            pl.BlockSpec((R, 3), lambda i: (i, 0)),
            pl.BlockSpec((3, H1), lambda i: (0, 0)),
            pl.BlockSpec((1, H1), lambda i: (0, 0)),
        ],
        out_specs=[
            pl.BlockSpec((R, C + H1 + 32), lambda i: (i, 0)),
            pl.BlockSpec((R, H1), lambda i: (i, 0)),
        ],
        out_shape=[
            jax.ShapeDtypeStruct((NB, C + H1 + 32), jnp.float32),
            jax.ShapeDtypeStruct((NB, H1), jnp.float32),
        ],
    )(features, input_pts, output_pts, l1r, c0)


def _sc_gather_body(table_hbm, idx_hbm, out_hbm, idx_v, buf0, buf1,
                    sem0, sem1, *, per_w, nc):
    wid = lax.axis_index("s") * nc + lax.axis_index("c")
    base = wid * per_w
    pltpu.sync_copy(idx_hbm.at[pl.ds(base, per_w)], idx_v)

    nch = per_w // _CH

    def start(k, buf, sem):
        pltpu.async_copy(
            table_hbm.at[idx_v.at[pl.ds(k * _CH, _CH)]], buf, sem)

    def drain(k, buf, sem):
        pltpu.make_async_copy(
            table_hbm.at[idx_v.at[pl.ds(k * _CH, _CH)]], buf, sem).wait()
        pltpu.sync_copy(buf, out_hbm.at[pl.ds(base + k * _CH, _CH)])

    start(0, buf0, sem0)
    hi = nch - 1 if nch % 2 == 1 else nch - 2

    @pl.loop(1, hi, step=2)
    def _pair(k):
        start(k, buf1, sem1)
        drain(k - 1, buf0, sem0)
        start(k + 1, buf0, sem0)
        drain(k, buf1, sem1)

    if nch % 2 == 1:
        drain(nch - 1, buf0, sem0)
    else:
        start(nch - 1, buf1, sem1)
        drain(nch - 2, buf0, sem0)
        drain(nch - 1, buf1, sem1)


def _sc_gather(table, idx, n_edges, width):
    info = plsc.get_sparse_core_info()
    nc, ns = info.num_cores, info.num_subcores
    nw = nc * ns
    per_w = n_edges // nw
    assert n_edges % nw == 0 and per_w % _CH == 0 and per_w % 8 == 0
    mesh = plsc.VectorSubcoreMesh(core_axis_name="c", subcore_axis_name="s")
    body = functools.partial(_sc_gather_body, per_w=per_w, nc=nc)
    f = pl.kernel(
        body,
        out_type=jax.ShapeDtypeStruct((n_edges, width), table.dtype),
        mesh=mesh,
        scratch_types=[
            pltpu.VMEM((per_w,), jnp.int32),
            pltpu.VMEM((_CH, width), table.dtype),
            pltpu.VMEM((_CH, width), table.dtype),
            pltpu.SemaphoreType.DMA,
            pltpu.SemaphoreType.DMA,
        ],
    )
    return f(table, idx)


def _block_kernel(g_ref, qq_ref, l2w_ref, l2b_ref, l3w_ref, l3b_ref,
                  wp_ref, bias_ref, out_ref, *, P, K, C, KSZ, OUT, H1):
    PK = P * K
    pp = g_ref[:, C:C + H1]
    qq = qq_ref[...]
    qq_rep = jnp.broadcast_to(qq[:, None, :], (P, K, H1)).reshape(PK, H1)
    h = jnp.maximum(pp - qq_rep, 0.0)
    h = jnp.maximum(jnp.dot(h, l2w_ref[...],
                            preferred_element_type=jnp.float32)
                    + l2b_ref[...], 0.0)
    d = jnp.maximum(jnp.dot(h, l3w_ref[...],
                            preferred_element_type=jnp.float32)
                    + l3b_ref[...], 0.0)

    feats2 = jnp.tile(g_ref[:, :C], (1, 2))
    row16 = lax.broadcasted_iota(jnp.int32, (KSZ, 2 * C), 0)
    half = lax.broadcasted_iota(jnp.int32, (KSZ, 2 * C), 1) // C
    acc8 = jnp.zeros((P * 8, OUT), dtype=jnp.float32)
    for i in range(KSZ // 2):
        sel = (row16 == 2 * i + half).astype(jnp.float32)
        dexp = jnp.dot(d, sel, preferred_element_type=jnp.float32)
        prod = feats2 * dexp
        s8 = prod.reshape(P, 2, 8, 2 * C).sum(axis=1).reshape(P * 8, 2 * C)
        acc8 = acc8 + jnp.dot(s8, wp_ref[i],
                              preferred_element_type=jnp.float32)
    out = acc8.reshape(P, 8, OUT).sum(axis=1) * (1.0 / K) + bias_ref[...]
    out_ref[...] = jnp.maximum(out, 0.0)


def kernel(features, input_pts, output_pts, W, bias, centers,
           l1_w, l1_b, l2_w, l2_b, l3_w, l3_b, indices_, neighbor_num):
    B, N, C = features.shape
    K = indices_.shape[2]
    KSZ = centers.shape[1]
    OUT = W.shape[2]
    H1 = l1_w.shape[0]
    NB = B * N
    WIDTH = C + H1 + 32

    l1w3 = l1_w.reshape(H1, 3, KSZ)
    l1r = jnp.sum(l1w3, axis=2).T
    c0 = (l1_b - jnp.einsum('mij,ij->m', l1w3, centers)).reshape(1, H1)

    offs = (jnp.arange(B, dtype=indices_.dtype) * N)[:, None, None]
    idx = (indices_ + offs).reshape(NB * K).astype(jnp.int32)

    table, qq = _build_table(features.reshape(NB, C), input_pts.reshape(NB, 3),
                             output_pts.reshape(NB, 3), l1r, c0, NB, C, H1)
    N0 = 26000
    E0 = N0 * K
    g0 = _sc_gather(table, idx[:E0], E0, WIDTH)
    g1 = _sc_gather(table, idx[E0:], NB * K - E0, WIDTH)

    P = 400
    assert N0 % P == 0 and (NB - N0) % P == 0

    wp = jnp.transpose(W, (1, 0, 2)).reshape(KSZ // 2, 2 * C, OUT)
    body = functools.partial(_block_kernel, P=P, K=K, C=C, KSZ=KSZ,
                             OUT=OUT, H1=H1)

    def dense(g_part, qq_part, npts):
        return pl.pallas_call(
            body,
            grid=(npts // P,),
            in_specs=[
                pl.BlockSpec((P * K, WIDTH), lambda i: (i, 0)),
                pl.BlockSpec((P, H1), lambda i: (i, 0)),
                pl.BlockSpec((2 * KSZ, KSZ), lambda i: (0, 0)),
                pl.BlockSpec((1, KSZ), lambda i: (0, 0)),
                pl.BlockSpec((KSZ, KSZ), lambda i: (0, 0)),
                pl.BlockSpec((1, KSZ), lambda i: (0, 0)),
                pl.BlockSpec((KSZ // 2, 2 * C, OUT), lambda i: (0, 0, 0)),
                pl.BlockSpec((1, OUT), lambda i: (0, 0)),
            ],
            out_specs=pl.BlockSpec((P, OUT), lambda i: (i, 0)),
            out_shape=jax.ShapeDtypeStruct((npts, OUT), jnp.float32),
        )(g_part, qq_part, l2_w.T, l2_b.reshape(1, -1), l3_w.T,
          l3_b.reshape(1, -1), wp, bias.reshape(1, -1))

    out0 = dense(g0, qq[:N0], N0)
    out1 = dense(g1, qq[N0:], NB - N0)
    return jnp.concatenate([out0, out1], axis=0).reshape(B, N, OUT)
